# baseline (speedup 1.0000x reference)
import functools

import jax
import jax.numpy as jnp
from jax import lax
from jax.experimental import pallas as pl
from jax.experimental.pallas import tpu as pltpu
from jax.experimental.pallas import tpu_sc as plsc


def _router_logits_body(x_ref, wr_ref, br_ref, out_ref):
    out_ref[...] = (
        jnp.dot(x_ref[...], wr_ref[...], preferred_element_type=jnp.float32)
        + br_ref[...]
    )


def _softmax_body(l_ref, p_ref, pT_ref, mask_ref, tval_ref, istar_ref, *, C):
    l = l_ref[...]
    B, E = l.shape
    m = jnp.max(l, axis=0, keepdims=True)
    ex = jnp.exp(l - m)
    s = jnp.sum(ex, axis=0, keepdims=True)
    p = ex / s
    p_ref[...] = p
    pT_ref[...] = p.T
    keys = jax.lax.bitcast_convert_type(p, jnp.int32)

    def bit_step(i, t):
        cand = t | (jnp.int32(1) << (30 - i))
        cnt = jnp.sum((keys >= cand).astype(jnp.int32), axis=0, keepdims=True)
        return jnp.where(cnt >= C, cand, t)

    t = jax.lax.fori_loop(0, 31, bit_step, jnp.zeros((1, E), jnp.int32))
    n_gt = jnp.sum((keys > t).astype(jnp.int32), axis=0, keepdims=True)
    rem = C - n_gt
    eq = keys == t
    iota = jax.lax.broadcasted_iota(jnp.int32, (B, 1), 0)

    def idx_step(i, s2):
        t2 = s2 + (jnp.int32(1) << (13 - i))
        cnt = jnp.sum((eq & (iota < t2)).astype(jnp.int32), axis=0,
                      keepdims=True)
        return jnp.where(cnt < rem, t2, s2)

    istar = jax.lax.fori_loop(0, 14, idx_step, jnp.zeros((1, E), jnp.int32))
    sel = (keys > t) | (eq & (iota <= istar))
    mask_ref[...] = sel.astype(jnp.float32)
    tval_ref[...] = t
    istar_ref[...] = istar


def _ffn_body(xs_ref, w1_ref, b1_ref, w2_ref, b2_ref, s_ref, out_ref):
    xs = xs_ref[0].astype(jnp.bfloat16)
    w1 = w1_ref[0].astype(jnp.bfloat16)
    h = jnp.dot(xs, w1, preferred_element_type=jnp.float32) + b1_ref[0]
    h = jnp.maximum(h, 0.0).astype(jnp.bfloat16)
    w2 = w2_ref[0].astype(jnp.bfloat16)
    y = jnp.dot(h, w2, preferred_element_type=jnp.float32) + b2_ref[0]
    out_ref[0] = y * s_ref[0, 0][:, None]


def _sc_scatter_add(idx_flat, ywf, B, O):
    EC = ywf.shape[0]
    NBUK = 16
    RNG = B // NBUK
    SH = RNG.bit_length() - 1
    PT = EC // 16
    CH = 64
    MAXCH = PT // CH
    TPP = RNG // 16
    mesh = plsc.VectorSubcoreMesh(core_axis_name="c", subcore_axis_name="s")

    @functools.partial(
        pl.kernel,
        mesh=mesh,
        compiler_params=pltpu.CompilerParams(needs_layout_passes=False),
        out_type=jax.ShapeDtypeStruct((B, 1, O), jnp.float32),
        scratch_types=[
            pltpu.VMEM((PT,), jnp.int32),
            pltpu.VMEM((PT + CH,), jnp.int32),
            pltpu.VMEM((MAXCH + 1, CH), jnp.int32),
            pltpu.VMEM((CH, 1, O), jnp.float32),
            pltpu.VMEM((16, 1, O), jnp.float32),
            pltpu.VMEM_SHARED((RNG + 16, 1, O), jnp.float32),
            pltpu.SemaphoreType.DMA,
        ],
    )
    def scatter_kernel(idx_hbm, ywf_hbm, out_hbm, idx_v, pos_flat, loc2d,
                       rows_v, zbuf, sp, sem):
        c = lax.axis_index("c")
        s = lax.axis_index("s")
        iota = lax.iota(jnp.int32, 16)
        zeros16 = jnp.zeros((16,), jnp.float32)

        def zb_body(i, _):
            zbuf[i // (O // 16), 0, pl.ds((i % (O // 16)) * 16, 16)] = zeros16
            return 0

        lax.fori_loop(0, 16 * (O // 16), zb_body, 0)
        pltpu.sync_copy(idx_hbm.at[pl.ds(s * PT, PT)], idx_v)

        def zero_slice():
            for m in range(TPP // 16):
                pltpu.sync_copy(zbuf, sp.at[pl.ds(s * TPP + m * 16, 16)])

        zero_slice()
        plsc.subcore_barrier()

        for p in range(NBUK // 2):
            bucket = 2 * p + c
            base = bucket * RNG

            def filt_body(j, off):
                v = idx_v[pl.ds(j * 16, 16)]
                m = (v >> SH) == bucket
                mi = m.astype(jnp.int32)
                pc = plsc.cumsum(mi)
                tgt = off + pc - 1
                gpos = s * PT + j * 16 + iota
                plsc.store_scatter(pos_flat, [tgt], gpos, mask=m)
                plsc.store_scatter(loc2d, [tgt >> 6, tgt & (CH - 1)],
                                   v & (RNG - 1), mask=m)
                return off + jnp.sum(mi)

            off = lax.fori_loop(0, PT // 16, filt_body, jnp.int32(0))

            ones = jnp.ones((16,), jnp.bool_)
            for m in range(CH // 16):
                tgt = off + m * 16 + iota
                plsc.store_scatter(pos_flat, [tgt],
                                   s * PT + m * 16 + iota, mask=ones)
                plsc.store_scatter(loc2d, [tgt >> 6, tgt & (CH - 1)],
                                   jnp.full((16,), RNG, jnp.int32) + s,
                                   mask=ones)

            nch = (off + CH - 1) // CH

            def chunk_body(k, _):
                pltpu.async_copy(
                    ywf_hbm.at[pos_flat.at[pl.ds(k * CH, CH)]], rows_v,
                    sem).wait()
                pltpu.sync_copy(rows_v, sp.at[loc2d.at[k]], add=True)
                return 0

            lax.fori_loop(0, nch, chunk_body, 0)
            plsc.subcore_barrier()

            pltpu.sync_copy(sp.at[pl.ds(s * TPP, TPP)],
                            rows_v.at[pl.ds(0, TPP)])
            pltpu.sync_copy(rows_v.at[pl.ds(0, TPP)],
                            out_hbm.at[pl.ds(base + s * TPP, TPP)])
            zero_slice()
            plsc.subcore_barrier()

    return scatter_kernel(idx_flat, ywf.reshape(EC, 1, O)).reshape(B, O)


def kernel(x, Wr, br, W1, b1, W2, b2):
    B, D = x.shape
    E = Wr.shape[1]
    H = W1.shape[2]
    O = W2.shape[2]
    C = min(512, B)

    RB = min(1024, B)
    logits = pl.pallas_call(
        _router_logits_body,
        grid=(B // RB,),
        in_specs=[
            pl.BlockSpec((RB, D), lambda i: (i, 0)),
            pl.BlockSpec((D, E), lambda i: (0, 0)),
            pl.BlockSpec((1, E), lambda i: (0, 0)),
        ],
        out_specs=pl.BlockSpec((RB, E), lambda i: (i, 0)),
        out_shape=jax.ShapeDtypeStruct((B, E), jnp.float32),
    )(x, Wr, br.reshape(1, E))

    probs, probsT, expert_mask, tval, istar = pl.pallas_call(
        functools.partial(_softmax_body, C=C),
        out_shape=(
            jax.ShapeDtypeStruct((B, E), jnp.float32),
            jax.ShapeDtypeStruct((E, B), jnp.float32),
            jax.ShapeDtypeStruct((B, E), jnp.float32),
            jax.ShapeDtypeStruct((1, E), jnp.int32),
            jax.ShapeDtypeStruct((1, E), jnp.int32),
        ),
    )(logits)

    top_scores, top_idx = jax.lax.top_k(probsT, C)

    xs = x[top_idx]

    y_w = pl.pallas_call(
        _ffn_body,
        grid=(E,),
        in_specs=[
            pl.BlockSpec((1, C, D), lambda e: (e, 0, 0)),
            pl.BlockSpec((1, D, H), lambda e: (e, 0, 0)),
            pl.BlockSpec((1, 1, H), lambda e: (e, 0, 0)),
            pl.BlockSpec((1, H, O), lambda e: (e, 0, 0)),
            pl.BlockSpec((1, 1, O), lambda e: (e, 0, 0)),
            pl.BlockSpec((1, 1, C), lambda e: (e, 0, 0)),
        ],
        out_specs=pl.BlockSpec((1, C, O), lambda e: (e, 0, 0)),
        out_shape=jax.ShapeDtypeStruct((E, C, O), jnp.float32),
    )(xs, W1, b1.reshape(E, 1, H), W2, b2.reshape(E, 1, O),
      top_scores.reshape(E, 1, C))

    out = _sc_scatter_add(top_idx.reshape(-1), y_w.reshape(E * C, O), B, O)
    return (out, probs, expert_mask)


# speedup vs baseline: 2.4682x; 2.4682x over previous
import functools

import jax
import jax.numpy as jnp
from jax import lax
from jax.experimental import pallas as pl
from jax.experimental.pallas import tpu as pltpu
from jax.experimental.pallas import tpu_sc as plsc


def _router_logits_body(x_ref, wr_ref, br_ref, out_ref):
    out_ref[...] = (
        jnp.dot(x_ref[...], wr_ref[...], preferred_element_type=jnp.float32)
        + br_ref[...]
    )


def _softmax_body(l_ref, p_ref, pT_ref, mask_ref, tval_ref, istar_ref, *, C):
    l = l_ref[...]
    B, E = l.shape
    m = jnp.max(l, axis=0, keepdims=True)
    ex = jnp.exp(l - m)
    s = jnp.sum(ex, axis=0, keepdims=True)
    p = ex / s
    p_ref[...] = p
    pT_ref[...] = p.T
    keys = jax.lax.bitcast_convert_type(p, jnp.int32)

    def bit_step(i, t):
        cand = t | (jnp.int32(1) << (30 - i))
        cnt = jnp.sum((keys >= cand).astype(jnp.int32), axis=0, keepdims=True)
        return jnp.where(cnt >= C, cand, t)

    t = jax.lax.fori_loop(0, 31, bit_step, jnp.zeros((1, E), jnp.int32))
    n_gt = jnp.sum((keys > t).astype(jnp.int32), axis=0, keepdims=True)
    rem = C - n_gt
    eq = keys == t
    iota = jax.lax.broadcasted_iota(jnp.int32, (B, 1), 0)

    def idx_step(i, s2):
        t2 = s2 + (jnp.int32(1) << (13 - i))
        cnt = jnp.sum((eq & (iota < t2)).astype(jnp.int32), axis=0,
                      keepdims=True)
        return jnp.where(cnt < rem, t2, s2)

    istar = jax.lax.fori_loop(0, 14, idx_step, jnp.zeros((1, E), jnp.int32))
    sel = (keys > t) | (eq & (iota <= istar))
    mask_ref[...] = sel.astype(jnp.float32)
    tval_ref[...] = t
    istar_ref[...] = istar


def _ffn_body(xs_ref, w1_ref, b1_ref, w2_ref, b2_ref, s_ref, out_ref):
    xs = xs_ref[...].astype(jnp.bfloat16)
    w1 = w1_ref[0].astype(jnp.bfloat16)
    h = jnp.dot(xs, w1, preferred_element_type=jnp.float32) + b1_ref[0]
    h = jnp.maximum(h, 0.0).astype(jnp.bfloat16)
    w2 = w2_ref[0].astype(jnp.bfloat16)
    y = jnp.dot(h, w2, preferred_element_type=jnp.float32) + b2_ref[0]
    out_ref[...] = (y * s_ref[0, 0][:, None])[:, None, :]


def _sc_compact_gather(probsT, tval, istar, x, C):
    E, B = probsT.shape
    D = x.shape[1]
    EC = E * C
    CH = 128
    NV = B // 16
    mesh = plsc.VectorSubcoreMesh(core_axis_name="c", subcore_axis_name="s")

    @functools.partial(
        pl.kernel,
        mesh=mesh,
        compiler_params=pltpu.CompilerParams(needs_layout_passes=False),
        out_type=(
            jax.ShapeDtypeStruct((EC,), jnp.int32),
            jax.ShapeDtypeStruct((EC, D), jnp.float32),
            jax.ShapeDtypeStruct((E, C), jnp.float32),
        ),
        scratch_types=[
            pltpu.VMEM((B,), jnp.float32),
            pltpu.VMEM((E,), jnp.int32),
            pltpu.VMEM((E,), jnp.int32),
            pltpu.VMEM((C,), jnp.int32),
            pltpu.VMEM((C,), jnp.float32),
            pltpu.VMEM((CH, D), jnp.float32),
            pltpu.SemaphoreType.DMA,
        ],
    )
    def cg_kernel(pT_hbm, tval_hbm, istar_hbm, x_hbm, idx_hbm, xs_hbm,
                  sc_hbm, prow, tv_v, is_v, idxbuf, scbuf, xrows, sem):
        c = lax.axis_index("c")
        s = lax.axis_index("s")
        w = s * 2 + c
        iota = lax.iota(jnp.int32, 16)
        pltpu.sync_copy(tval_hbm.at[0], tv_v)
        pltpu.sync_copy(istar_hbm.at[0], is_v)

        for t in range(2):
            e = w * 2 + t
            pltpu.sync_copy(pT_hbm.at[e], prow)
            esplat = jnp.zeros((16,), jnp.int32) + e
            tv = plsc.load_gather(tv_v, [esplat])
            isv = plsc.load_gather(is_v, [esplat])

            def scan_body(j, off):
                fv = prow[pl.ds(j * 16, 16)]
                v = plsc.bitcast(fv, jnp.int32)
                gidx = j * 16 + iota
                m = (v > tv) | ((v == tv) & (gidx <= isv))
                pc = plsc.cumsum(m.astype(jnp.int32))
                tgt = off + pc - 1
                plsc.store_scatter(idxbuf, [tgt], gidx, mask=m)
                plsc.store_scatter(scbuf, [tgt], fv, mask=m)
                return off + jnp.sum(m.astype(jnp.int32))

            lax.fori_loop(0, NV, scan_body, jnp.int32(0))
            pltpu.sync_copy(idxbuf, idx_hbm.at[pl.ds(e * C, C)])
            pltpu.sync_copy(scbuf, sc_hbm.at[e])
            for k in range(C // CH):
                pltpu.async_copy(
                    x_hbm.at[idxbuf.at[pl.ds(k * CH, CH)]], xrows, sem).wait()
                pltpu.sync_copy(
                    xrows, xs_hbm.at[pl.ds(e * C + k * CH, CH)])

    return cg_kernel(probsT, tval, istar, x)


def _sc_scatter_add(idx_flat, ywf, B, O):
    EC = ywf.shape[0]
    NBUK = 16
    RNG = B // NBUK
    SH = RNG.bit_length() - 1
    PT = EC // 16
    CH = 64
    MAXCH = PT // CH
    TPP = RNG // 16
    mesh = plsc.VectorSubcoreMesh(core_axis_name="c", subcore_axis_name="s")

    @functools.partial(
        pl.kernel,
        mesh=mesh,
        compiler_params=pltpu.CompilerParams(needs_layout_passes=False),
        out_type=jax.ShapeDtypeStruct((B, 1, O), jnp.float32),
        scratch_types=[
            pltpu.VMEM((PT,), jnp.int32),
            pltpu.VMEM((PT + CH,), jnp.int32),
            pltpu.VMEM((MAXCH + 1, CH), jnp.int32),
            pltpu.VMEM((CH, 1, O), jnp.float32),
            pltpu.VMEM((16, 1, O), jnp.float32),
            pltpu.VMEM_SHARED((RNG + 16, 1, O), jnp.float32),
            pltpu.SemaphoreType.DMA,
        ],
    )
    def scatter_kernel(idx_hbm, ywf_hbm, out_hbm, idx_v, pos_flat, loc2d,
                       rows_v, zbuf, sp, sem):
        c = lax.axis_index("c")
        s = lax.axis_index("s")
        iota = lax.iota(jnp.int32, 16)
        zeros16 = jnp.zeros((16,), jnp.float32)

        def zb_body(i, _):
            zbuf[i // (O // 16), 0, pl.ds((i % (O // 16)) * 16, 16)] = zeros16
            return 0

        lax.fori_loop(0, 16 * (O // 16), zb_body, 0)
        pltpu.sync_copy(idx_hbm.at[pl.ds(s * PT, PT)], idx_v)

        def zero_slice():
            for m in range(TPP // 16):
                pltpu.sync_copy(zbuf, sp.at[pl.ds(s * TPP + m * 16, 16)])

        zero_slice()
        plsc.subcore_barrier()

        for p in range(NBUK // 2):
            bucket = 2 * p + c
            base = bucket * RNG

            def filt_body(j, off):
                v = idx_v[pl.ds(j * 16, 16)]
                m = (v >> SH) == bucket
                mi = m.astype(jnp.int32)
                pc = plsc.cumsum(mi)
                tgt = off + pc - 1
                gpos = s * PT + j * 16 + iota
                plsc.store_scatter(pos_flat, [tgt], gpos, mask=m)
                plsc.store_scatter(loc2d, [tgt >> 6, tgt & (CH - 1)],
                                   v & (RNG - 1), mask=m)
                return off + jnp.sum(mi)

            off = lax.fori_loop(0, PT // 16, filt_body, jnp.int32(0))

            ones = jnp.ones((16,), jnp.bool_)
            for m in range(CH // 16):
                tgt = off + m * 16 + iota
                plsc.store_scatter(pos_flat, [tgt],
                                   s * PT + m * 16 + iota, mask=ones)
                plsc.store_scatter(loc2d, [tgt >> 6, tgt & (CH - 1)],
                                   jnp.full((16,), RNG, jnp.int32) + s,
                                   mask=ones)

            nch = (off + CH - 1) // CH

            def chunk_body(k, _):
                pltpu.async_copy(
                    ywf_hbm.at[pos_flat.at[pl.ds(k * CH, CH)]], rows_v,
                    sem).wait()
                pltpu.sync_copy(rows_v, sp.at[loc2d.at[k]], add=True)
                return 0

            lax.fori_loop(0, nch, chunk_body, 0)
            plsc.subcore_barrier()

            pltpu.sync_copy(sp.at[pl.ds(s * TPP, TPP)],
                            rows_v.at[pl.ds(0, TPP)])
            pltpu.sync_copy(rows_v.at[pl.ds(0, TPP)],
                            out_hbm.at[pl.ds(base + s * TPP, TPP)])
            zero_slice()
            plsc.subcore_barrier()

    return scatter_kernel(idx_flat, ywf).reshape(B, O)


def kernel(x, Wr, br, W1, b1, W2, b2):
    B, D = x.shape
    E = Wr.shape[1]
    H = W1.shape[2]
    O = W2.shape[2]
    C = min(512, B)

    RB = min(1024, B)
    logits = pl.pallas_call(
        _router_logits_body,
        grid=(B // RB,),
        in_specs=[
            pl.BlockSpec((RB, D), lambda i: (i, 0)),
            pl.BlockSpec((D, E), lambda i: (0, 0)),
            pl.BlockSpec((1, E), lambda i: (0, 0)),
        ],
        out_specs=pl.BlockSpec((RB, E), lambda i: (i, 0)),
        out_shape=jax.ShapeDtypeStruct((B, E), jnp.float32),
    )(x, Wr, br.reshape(1, E))

    probs, probsT, expert_mask, tval, istar = pl.pallas_call(
        functools.partial(_softmax_body, C=C),
        out_shape=(
            jax.ShapeDtypeStruct((B, E), jnp.float32),
            jax.ShapeDtypeStruct((E, B), jnp.float32),
            jax.ShapeDtypeStruct((B, E), jnp.float32),
            jax.ShapeDtypeStruct((1, E), jnp.int32),
            jax.ShapeDtypeStruct((1, E), jnp.int32),
        ),
    )(logits)

    top_idx_flat, xs, top_scores = _sc_compact_gather(probsT, tval, istar, x, C)

    y_w = pl.pallas_call(
        _ffn_body,
        grid=(E,),
        in_specs=[
            pl.BlockSpec((C, D), lambda e: (e, 0)),
            pl.BlockSpec((1, D, H), lambda e: (e, 0, 0)),
            pl.BlockSpec((1, 1, H), lambda e: (e, 0, 0)),
            pl.BlockSpec((1, H, O), lambda e: (e, 0, 0)),
            pl.BlockSpec((1, 1, O), lambda e: (e, 0, 0)),
            pl.BlockSpec((1, 1, C), lambda e: (e, 0, 0)),
        ],
        out_specs=pl.BlockSpec((C, 1, O), lambda e: (e, 0, 0)),
        out_shape=jax.ShapeDtypeStruct((E * C, 1, O), jnp.float32),
    )(xs, W1, b1.reshape(E, 1, H), W2, b2.reshape(E, 1, O),
      top_scores.reshape(E, 1, C))

    out = _sc_scatter_add(top_idx_flat, y_w, B, O)
    return (out, probs, expert_mask)
